# baseline (speedup 1.0000x reference)
import jax
import jax.numpy as jnp
from jax import lax
from jax.experimental import pallas as pl
from jax.experimental.pallas import tpu as pltpu
from jax.experimental.pallas import tpu_sc as plsc

NC = 2
NS = 16
L = 16
NW = NC * NS
D = 64
SEQ = 200
CHUNK = 160
GPC = CHUNK // L
KD = D // L
PROWS = 180

_GDN = lax.GatherDimensionNumbers(
    offset_dims=(), collapsed_slice_dims=(0,), start_index_map=(0,))


def _lane_splat(vec, i):
    return lax.gather(vec, jnp.full((L, 1), i, jnp.int32), _GDN,
                      slice_sizes=(1,), mode=lax.GatherScatterMode.PROMISE_IN_BOUNDS)


def _rsqrt(w):
    yi = jnp.int32(0x5F3759DF) - (plsc.bitcast(w, jnp.int32) >> 1)
    y = plsc.bitcast(yi, jnp.float32)
    for _ in range(3):
        y = y * (1.5 - 0.5 * w * y * y)
    return y


def _body(xh, par_h, sf, tok_hbm, posp_hbm, dseg_hbm, gam_hbm, bet_hbm,
          out_hbm, idx0, idx1, sv_v, par_v, tok0, tok1, posp_v, dseg_v, gb_v,
          mean_v, r_v, out_v, gsem0, gsem1, osem):
    wid = lax.axis_index("s") * NC + lax.axis_index("c")
    n_chunks = (out_hbm.shape[0] * 2) // (NW * CHUNK)
    wbase = wid * (n_chunks * CHUNK)

    pltpu.sync_copy(posp_hbm, posp_v)
    pltpu.sync_copy(dseg_hbm, dseg_v)
    pltpu.sync_copy(gam_hbm, gb_v.at[0])
    pltpu.sync_copy(bet_hbm, gb_v.at[1])
    gks = [gb_v[0, pl.ds(k * L, L)] for k in range(KD)]
    bks = [gb_v[1, pl.ds(k * L, L)] for k in range(KD)]
    dregs = [dseg_v[pl.ds(L + k * L, L)] for k in range(KD)]
    pparity = lax.rem(lax.iota(jnp.int32, 16), 2) * D

    def compute(c, tv, seq):
        base = pl.multiple_of(wbase + c * CHUNK, CHUNK)
        phase = lax.rem(c * CHUNK, SEQ)
        pltpu.sync_copy(sf.at[pl.ds(base, CHUNK)], sv_v)
        pltpu.sync_copy(par_h.at[pl.ds(base, CHUNK)], par_v)

        def pass_a(g, _):
            t0 = g * L
            ti = lax.iota(jnp.int32, 16) + t0
            prow = (lax.iota(jnp.int32, 16) + (t0 + phase)) >> 1
            svf = sv_v[pl.ds(t0, L)].astype(jnp.float32)
            pv64 = par_v[pl.ds(t0, L)]
            sp = [jnp.zeros((L,), jnp.float32) for _ in range(4)]
            qp = [jnp.zeros((L,), jnp.float32) for _ in range(4)]
            for w in range(D // 8):
                dd = range(w * 8, w * 8 + 8)
                tvs = [plsc.load_gather(tv, [ti, pv64 + d]) for d in dd]
                pvs = [plsc.load_gather(posp_v, [prow, pparity + d]) for d in dd]
                dvs = [_lane_splat(dregs[d // L], d % L) for d in dd]
                es = [t + p + dv * svf for t, p, dv in zip(tvs, pvs, dvs)]
                for j, d in enumerate(dd):
                    plsc.store_scatter(tv, [ti, pv64 + d], es[j])
                for j in range(8):
                    sp[j % 4] = sp[j % 4] + es[j]
                    qp[j % 4] = qp[j % 4] + es[j] * es[j]
            s = (sp[0] + sp[1]) + (sp[2] + sp[3])
            q = (qp[0] + qp[1]) + (qp[2] + qp[3])
            mean = s * (1.0 / D)
            var = q * (1.0 / D) - mean * mean
            r = _rsqrt(var + 1e-5)
            mean_v[pl.ds(t0, L)] = mean
            r_v[pl.ds(t0, L)] = r
            return 0

        lax.fori_loop(0, GPC, pass_a, 0)

        def pass_b(g, _):
            t0 = g * L
            m16 = mean_v[pl.ds(t0, L)]
            r16 = r_v[pl.ds(t0, L)]
            p16 = par_v[pl.ds(t0, L)]
            for tw in range(4):
                tls = range(tw * 4, tw * 4 + 4)
                msps = [_lane_splat(m16, tl) for tl in tls]
                rsps = [_lane_splat(r16, tl) for tl in tls]
                pars = [p16[tl] for tl in tls]
                evs = [[tv[t0 + tl, pl.ds(pars[j] + k * L, L)]
                        for k in range(KD)] for j, tl in enumerate(tls)]
                for j, tl in enumerate(tls):
                    for k in range(KD):
                        z = (evs[j][k] - msps[j]) * rsps[j]
                        out_v[g * 8 + tl // 2,
                              pl.ds((tl % 2) * D + k * L, L)] = (
                                  z * gks[k] + bks[k])
            return 0

        @pl.when(seq > 0)
        def _():
            pltpu.make_async_copy(
                out_v, out_hbm.at[pl.ds(pl.multiple_of((base - CHUNK) // 2, 8), CHUNK // 2)],
                osem).wait()

        lax.fori_loop(0, GPC, pass_b, 0)
        pltpu.async_copy(out_v, out_hbm.at[pl.ds(pl.multiple_of(base // 2, 8), CHUNK // 2)], osem)

    NSUB = 2
    SUB = CHUNK // NSUB

    def issue_gather(idx, tv, sem):
        for i in range(NSUB):
            pltpu.async_copy(tok_hbm.at[idx.at[pl.ds(i * SUB, SUB)]],
                             tv.at[pl.ds(i * SUB, SUB)], sem)

    def wait_gather(idx, tv, sem):
        for i in range(NSUB):
            pltpu.make_async_copy(tok_hbm.at[idx.at[pl.ds(i * SUB, SUB)]],
                                  tv.at[pl.ds(i * SUB, SUB)], sem).wait()

    pltpu.sync_copy(xh.at[pl.ds(wbase, CHUNK)], idx0)
    issue_gather(idx0, tok0, gsem0)

    def pair_body(cp, _):
        c0 = 2 * cp
        pltpu.sync_copy(xh.at[pl.ds(wbase + (c0 + 1) * CHUNK, CHUNK)], idx1)
        issue_gather(idx1, tok1, gsem1)
        wait_gather(idx0, tok0, gsem0)
        compute(c0, tok0, 2 * cp)

        @pl.when(cp < n_chunks // 2 - 1)
        def _():
            pltpu.sync_copy(xh.at[pl.ds(wbase + (c0 + 2) * CHUNK, CHUNK)], idx0)
            issue_gather(idx0, tok0, gsem0)

        wait_gather(idx1, tok1, gsem1)
        compute(c0 + 1, tok1, 1)
        return 0

    lax.fori_loop(0, n_chunks // 2, pair_body, 0)
    pltpu.make_async_copy(
        out_v,
        out_hbm.at[pl.ds(pl.multiple_of(
            (wbase + (n_chunks - 1) * CHUNK) // 2, 8), CHUNK // 2)],
        osem).wait()


def kernel(x, seg, token_table, pos_table, seg_table, gamma, beta):
    B, S = x.shape
    V, d_model = token_table.shape
    assert d_model == D and S == SEQ and (B * S) % (NW * CHUNK) == 0

    xf = x.reshape(-1)
    xh = xf >> 1
    par = (xf & 1) * D
    sf = seg.reshape(-1)
    tok2 = token_table.reshape(V // 2, 2 * D)
    pos2 = jnp.tile(pos_table, (2, 1)) + seg_table[0]
    posp = pos2[:2 * PROWS].reshape(PROWS, 2 * D)
    dseg = jnp.concatenate([jnp.zeros((L,), jnp.float32),
                            seg_table[1] - seg_table[0]])

    mesh = plsc.VectorSubcoreMesh(
        core_axis_name="c", subcore_axis_name="s",
        num_cores=NC, num_subcores=NS)

    call = pl.kernel(
        _body,
        out_type=jax.ShapeDtypeStruct((B * S // 2, 2 * D), jnp.float32),
        mesh=mesh,
        compiler_params=pltpu.CompilerParams(
            needs_layout_passes=False, use_tc_tiling_on_sc=True),
        scratch_types=[
            pltpu.VMEM((CHUNK,), jnp.int32),
            pltpu.VMEM((CHUNK,), jnp.int32),
            pltpu.VMEM((CHUNK,), jnp.int32),
            pltpu.VMEM((CHUNK,), jnp.int32),
            pltpu.VMEM((CHUNK, 2 * D), jnp.float32),
            pltpu.VMEM((CHUNK, 2 * D), jnp.float32),
            pltpu.VMEM((PROWS, 2 * D), jnp.float32),
            pltpu.VMEM((L + D,), jnp.float32),
            pltpu.VMEM((2, D), jnp.float32),
            pltpu.VMEM((CHUNK,), jnp.float32),
            pltpu.VMEM((CHUNK,), jnp.float32),
            pltpu.VMEM((CHUNK // 2, 2 * D), jnp.float32),
            pltpu.SemaphoreType.DMA,
            pltpu.SemaphoreType.DMA,
            pltpu.SemaphoreType.DMA,
        ],
    )
    out = call(xh, par, sf, tok2, posp, dseg, gamma, beta)
    return out.reshape(B, S, D)


# speedup vs baseline: 1.0434x; 1.0434x over previous
import jax
import jax.numpy as jnp
from jax import lax
from jax.experimental import pallas as pl
from jax.experimental.pallas import tpu as pltpu
from jax.experimental.pallas import tpu_sc as plsc

NC = 2
NS = 16
L = 16
NW = NC * NS
D = 64
CHUNK = 400
GPC = CHUNK // L
KD = D // L

_GDN = lax.GatherDimensionNumbers(
    offset_dims=(), collapsed_slice_dims=(0,), start_index_map=(0,))


def _lane_splat(vec, i):
    return lax.gather(vec, jnp.full((L, 1), i, jnp.int32), _GDN,
                      slice_sizes=(1,), mode=lax.GatherScatterMode.PROMISE_IN_BOUNDS)


def _csplat(v):
    return jnp.full((L,), v, jnp.int32)


def _rsqrt(w):
    yi = jnp.int32(0x5F3759DF) - (plsc.bitcast(w, jnp.int32) >> 1)
    y = plsc.bitcast(yi, jnp.float32)
    for _ in range(3):
        y = y * (1.5 - 0.5 * w * y * y)
    return y


def _body(xf, sf, tok_hbm, pos2_hbm, dseg_hbm, gam_hbm, bet_hbm, out_hbm,
          idx0, idx1, sv_v, tok0, tok1, pos2_v, dseg_v, gb_v, mean_v, r_v,
          out_v, gsem0, gsem1, osem):
    wid = lax.axis_index("s") * NC + lax.axis_index("c")
    n_chunks = out_hbm.shape[0] // (NW * CHUNK)
    wbase = wid * (n_chunks * CHUNK)

    pltpu.sync_copy(pos2_hbm, pos2_v)
    pltpu.sync_copy(dseg_hbm, dseg_v)
    pltpu.sync_copy(gam_hbm, gb_v.at[0])
    pltpu.sync_copy(bet_hbm, gb_v.at[1])
    gks = [gb_v[0, pl.ds(k * L, L)] for k in range(KD)]
    bks = [gb_v[1, pl.ds(k * L, L)] for k in range(KD)]
    dregs = [dseg_v[pl.ds(L + k * L, L)] for k in range(KD)]

    def compute(c, tv, seq):
        base = wbase + c * CHUNK
        pltpu.sync_copy(sf.at[pl.ds(base, CHUNK)], sv_v)

        def pass_a(g, _):
            t0 = g * L
            ti = lax.iota(jnp.int32, 16) + t0
            svf = sv_v[pl.ds(t0, L)].astype(jnp.float32)
            sp = [jnp.zeros((L,), jnp.float32) for _ in range(4)]
            qp = [jnp.zeros((L,), jnp.float32) for _ in range(4)]
            for w in range(D // 8):
                dd = range(w * 8, w * 8 + 8)
                tvs = [plsc.load_gather(tv, [ti, _csplat(d)]) for d in dd]
                pvs = [plsc.load_gather(pos2_v, [ti, _csplat(d)]) for d in dd]
                dvs = [_lane_splat(dregs[d // L], d % L) for d in dd]
                es = [t + p + dv * svf for t, p, dv in zip(tvs, pvs, dvs)]
                for j, d in enumerate(dd):
                    plsc.store_scatter(tv, [ti, _csplat(d)], es[j])
                for j in range(8):
                    sp[j % 4] = sp[j % 4] + es[j]
                    qp[j % 4] = qp[j % 4] + es[j] * es[j]
            s = (sp[0] + sp[1]) + (sp[2] + sp[3])
            q = (qp[0] + qp[1]) + (qp[2] + qp[3])
            mean = s * (1.0 / D)
            var = q * (1.0 / D) - mean * mean
            r = _rsqrt(var + 1e-5)
            mean_v[pl.ds(t0, L)] = mean
            r_v[pl.ds(t0, L)] = r
            return 0

        lax.fori_loop(0, GPC, pass_a, 0)

        def pass_b(g, _):
            t0 = g * L
            m16 = mean_v[pl.ds(t0, L)]
            r16 = r_v[pl.ds(t0, L)]
            for tw in range(4):
                tls = range(tw * 4, tw * 4 + 4)
                msps = [_lane_splat(m16, tl) for tl in tls]
                rsps = [_lane_splat(r16, tl) for tl in tls]
                evs = [[tv[t0 + tl, pl.ds(k * L, L)] for k in range(KD)]
                       for tl in tls]
                for j, tl in enumerate(tls):
                    for k in range(KD):
                        z = (evs[j][k] - msps[j]) * rsps[j]
                        out_v[t0 + tl, pl.ds(k * L, L)] = z * gks[k] + bks[k]
            return 0

        @pl.when(seq > 0)
        def _():
            pltpu.make_async_copy(
                out_v, out_hbm.at[pl.ds(base - CHUNK, CHUNK)], osem).wait()

        lax.fori_loop(0, GPC, pass_b, 0)
        pltpu.async_copy(out_v, out_hbm.at[pl.ds(base, CHUNK)], osem)

    NSUB = 5
    SUB = CHUNK // NSUB

    def issue_gather(idx, tv, sem):
        for i in range(NSUB):
            pltpu.async_copy(tok_hbm.at[idx.at[pl.ds(i * SUB, SUB)]],
                             tv.at[pl.ds(i * SUB, SUB)], sem)

    def wait_gather(idx, tv, sem):
        for i in range(NSUB):
            pltpu.make_async_copy(tok_hbm.at[idx.at[pl.ds(i * SUB, SUB)]],
                                  tv.at[pl.ds(i * SUB, SUB)], sem).wait()

    pltpu.sync_copy(xf.at[pl.ds(wbase, CHUNK)], idx0)
    issue_gather(idx0, tok0, gsem0)

    def pair_body(cp, _):
        c0 = 2 * cp
        pltpu.sync_copy(xf.at[pl.ds(wbase + (c0 + 1) * CHUNK, CHUNK)], idx1)
        issue_gather(idx1, tok1, gsem1)
        wait_gather(idx0, tok0, gsem0)
        compute(c0, tok0, 2 * cp)

        @pl.when(cp < n_chunks // 2 - 1)
        def _():
            pltpu.sync_copy(xf.at[pl.ds(wbase + (c0 + 2) * CHUNK, CHUNK)], idx0)
            issue_gather(idx0, tok0, gsem0)

        wait_gather(idx1, tok1, gsem1)
        compute(c0 + 1, tok1, 1)
        return 0

    lax.fori_loop(0, n_chunks // 2, pair_body, 0)
    pltpu.make_async_copy(
        out_v, out_hbm.at[pl.ds(wbase + (n_chunks - 1) * CHUNK, CHUNK)],
        osem).wait()


def kernel(x, seg, token_table, pos_table, seg_table, gamma, beta):
    B, S = x.shape
    V, d_model = token_table.shape
    assert d_model == D and (B * S) % (NW * CHUNK) == 0 and CHUNK % S == 0

    xf = x.reshape(-1)
    sf = seg.reshape(-1)
    reps = CHUNK // S
    pos2 = jnp.tile(pos_table, (reps, 1)) + seg_table[0]
    dseg = jnp.concatenate([jnp.zeros((L,), jnp.float32),
                            seg_table[1] - seg_table[0]])

    mesh = plsc.VectorSubcoreMesh(
        core_axis_name="c", subcore_axis_name="s",
        num_cores=NC, num_subcores=NS)

    call = pl.kernel(
        _body,
        out_type=jax.ShapeDtypeStruct((B * S, D), jnp.float32),
        mesh=mesh,
        compiler_params=pltpu.CompilerParams(
            needs_layout_passes=False, use_tc_tiling_on_sc=False),
        scratch_types=[
            pltpu.VMEM((CHUNK,), jnp.int32),
            pltpu.VMEM((CHUNK,), jnp.int32),
            pltpu.VMEM((CHUNK,), jnp.int32),
            pltpu.VMEM((CHUNK, D), jnp.float32),
            pltpu.VMEM((CHUNK, D), jnp.float32),
            pltpu.VMEM((CHUNK, D), jnp.float32),
            pltpu.VMEM((L + D,), jnp.float32),
            pltpu.VMEM((2, D), jnp.float32),
            pltpu.VMEM((CHUNK,), jnp.float32),
            pltpu.VMEM((CHUNK,), jnp.float32),
            pltpu.VMEM((CHUNK, D), jnp.float32),
            pltpu.SemaphoreType.DMA,
            pltpu.SemaphoreType.DMA,
            pltpu.SemaphoreType.DMA,
        ],
    )
    out = call(xf, sf, token_table, pos2, dseg, gamma, beta)
    return out.reshape(B, S, D)
